# Initial kernel scaffold; baseline (speedup 1.0000x reference)
import jax
import jax.numpy as jnp
from jax.experimental import pallas as pl


def kernel(x, edge_index, W0, b0, ln0_w, ln0_b, bn0_w, bn0_b, W1, b1):
    raise NotImplementedError("write your pallas kernel here")


# speedup vs baseline: 2.2944x; 2.2944x over previous
import functools

import jax
import jax.numpy as jnp
from jax import lax
from jax.experimental import pallas as pl
from jax.experimental.pallas import tpu as pltpu
from jax.experimental.pallas import tpu_sc as plsc

EPS = 1e-5


def _node_mm_body(x_ref, w_ref, b0_ref, ya_ref, yb_ref):
    d = x_ref.shape[1]
    xv = x_ref[...]
    ya_ref[...] = jnp.dot(xv, w_ref[:d, :], preferred_element_type=jnp.float32)
    yb_ref[...] = (
        jnp.dot(xv, w_ref[d:, :], preferred_element_type=jnp.float32)
        + b0_ref[...]
    )


def _node_mm(x, w0, b0):
    n, _ = x.shape
    h = w0.shape[1]
    out = jax.ShapeDtypeStruct((n, h), jnp.float32)
    return pl.pallas_call(
        _node_mm_body,
        out_shape=(out, out),
    )(x, w0, b0.reshape(1, h))


def _make_gather(rows_total, chunk, h, nc, ns):
    nw = nc * ns
    iters = (rows_total + nw - 1) // nw
    mesh = plsc.VectorSubcoreMesh(core_axis_name="c", subcore_axis_name="s")

    @functools.partial(
        pl.kernel,
        out_type=jax.ShapeDtypeStruct((rows_total, chunk, h), jnp.float32),
        mesh=mesh,
        scratch_types=[
            pltpu.VMEM((chunk,), jnp.int32),
            pltpu.VMEM((chunk,), jnp.int32),
            pltpu.VMEM((chunk, h), jnp.float32),
            pltpu.SemaphoreType.DMA,
        ],
        compiler_params=pltpu.CompilerParams(use_tc_tiling_on_sc=False),
    )
    def gather_kernel(ya_hbm, yb_hbm, s_hbm, e_hbm, g_hbm, idx_s, idx_e, rows, sem):
        wid = lax.axis_index("s") * nc + lax.axis_index("c")

        def body(j, carry):
            row = wid + nw * j

            @pl.when(row < rows_total)
            def _():
                pltpu.sync_copy(s_hbm.at[row], idx_s)
                pltpu.sync_copy(e_hbm.at[row], idx_e)
                pltpu.async_copy(ya_hbm.at[idx_s], rows, sem).wait()
                pltpu.async_copy(yb_hbm.at[idx_e], rows, sem, add=True).wait()
                pltpu.sync_copy(rows, g_hbm.at[row])

            return carry

        lax.fori_loop(0, iters, body, 0)

    return gather_kernel


def _stats_body(g_ref, lnw_ref, lnb_ref, o_ref):
    i = pl.program_id(0)
    hv = g_ref[...]
    m = jnp.mean(hv, axis=1, keepdims=True)
    v = jnp.mean((hv - m) ** 2, axis=1, keepdims=True)
    hln = (hv - m) / jnp.sqrt(v + EPS) * lnw_ref[...] + lnb_ref[...]
    ps = jnp.sum(hln, axis=0)
    ps2 = jnp.sum(hln * hln, axis=0)
    blk = jnp.stack([ps, ps2])

    @pl.when(i == 0)
    def _():
        o_ref[...] = blk

    @pl.when(i > 0)
    def _():
        o_ref[...] += blk


def _stats(g, lnw, lnb, blk_rows):
    e, h = g.shape
    nb = e // blk_rows
    return pl.pallas_call(
        _stats_body,
        grid=(nb,),
        in_specs=[
            pl.BlockSpec((blk_rows, h), lambda i: (i, 0)),
            pl.BlockSpec((1, h), lambda i: (0, 0)),
            pl.BlockSpec((1, h), lambda i: (0, 0)),
        ],
        out_specs=pl.BlockSpec((2, h), lambda i: (0, 0)),
        out_shape=jax.ShapeDtypeStruct((2, h), jnp.float32),
    )(g, lnw.reshape(1, h), lnb.reshape(1, h))


def _apply_body(g_ref, stats_ref, lnw_ref, lnb_ref, bnw_ref, bnb_ref,
                w1_ref, b1_ref, n_edges_ref, o_ref):
    hv = g_ref[...]
    m = jnp.mean(hv, axis=1, keepdims=True)
    v = jnp.mean((hv - m) ** 2, axis=1, keepdims=True)
    hln = (hv - m) / jnp.sqrt(v + EPS) * lnw_ref[...] + lnb_ref[...]

    inv_e = 1.0 / n_edges_ref[0]
    bmean = stats_ref[0:1, :] * inv_e
    bvar = stats_ref[1:2, :] * inv_e - bmean * bmean
    hbn = (hln - bmean) / jnp.sqrt(bvar + EPS) * bnw_ref[...] + bnb_ref[...]

    s = hbn * jax.nn.sigmoid(hbn)
    o_ref[...] = (
        jnp.dot(s, w1_ref[...], preferred_element_type=jnp.float32)
        + b1_ref[...]
    )


def _apply(g, stats, lnw, lnb, bnw, bnb, w1, b1, blk_rows):
    e, h = g.shape
    nb = e // blk_rows
    full = lambda i: (0, 0)
    return pl.pallas_call(
        _apply_body,
        grid=(nb,),
        in_specs=[
            pl.BlockSpec((blk_rows, h), lambda i: (i, 0)),
            pl.BlockSpec((2, h), full),
            pl.BlockSpec((1, h), full),
            pl.BlockSpec((1, h), full),
            pl.BlockSpec((1, h), full),
            pl.BlockSpec((1, h), full),
            pl.BlockSpec((h, 1), full),
            pl.BlockSpec((1, 1), full),
            pl.BlockSpec(memory_space=pltpu.SMEM),
        ],
        out_specs=pl.BlockSpec((blk_rows, 1), lambda i: (i, 0)),
        out_shape=jax.ShapeDtypeStruct((e, 1), jnp.float32),
    )(g, stats, lnw.reshape(1, h), lnb.reshape(1, h), bnw.reshape(1, h),
      bnb.reshape(1, h), w1, b1.reshape(1, 1),
      jnp.full((1,), float(e), jnp.float32))


def kernel(x, edge_index, W0, b0, ln0_w, ln0_b, bn0_w, bn0_b, W1, b1):
    n, d = x.shape
    e = edge_index.shape[1]
    h = W0.shape[1]
    chunk = 128
    rows_total = e // chunk

    start = edge_index[0].astype(jnp.int32).reshape(rows_total, chunk)
    end = edge_index[1].astype(jnp.int32).reshape(rows_total, chunk)

    ya, yb = _node_mm(x, W0, b0)

    info = plsc.get_sparse_core_info()
    g3 = _make_gather(rows_total, chunk, h, info.num_cores, info.num_subcores)(
        ya, yb, start, end
    )
    g = g3.reshape(e, h)

    blk_rows = 16000
    stats = _stats(g, ln0_w, ln0_b, blk_rows)
    out = _apply(g, stats, ln0_w, ln0_b, bn0_w, bn0_b, W1, b1, blk_rows)
    return jnp.squeeze(out, -1)
